# baseline (speedup 1.0000x reference)
import functools

import jax
import jax.numpy as jnp
from jax import lax
from jax.experimental import pallas as pl
from jax.experimental.pallas import tpu as pltpu
from jax.experimental.pallas import tpu_sc as plsc

N = 10000
E = 320000
D_IN = 128
D_HID = 256
DH = 128
NUM_LAYER = 5
NC = 2
NS = 16
CHUNK = 80
SEG_CHUNKS = E // NS // CHUNK
SEG_PHASES = 10
CPP = SEG_CHUNKS // SEG_PHASES
DEG_CHUNKS = E // (NC * NS) // CHUNK
ZR = 125

_mesh = plsc.VectorSubcoreMesh(
    core_axis_name="c", subcore_axis_name="s", num_cores=NC, num_subcores=NS)


def _fill_zeros_2d(ref, rows, cols):
    z = jnp.zeros((16,), jnp.float32)

    def row_body(r, _):
        def col_body(j, _):
            ref[r, pl.ds(j * 16, 16)] = z
            return 0
        return lax.fori_loop(0, cols // 16, col_body, 0)

    lax.fori_loop(0, rows, row_body, 0)


def _fill_1d(ref, n, value):
    v = jnp.full((16,), value, jnp.float32)

    def body(j, _):
        ref[pl.ds(j * 16, 16)] = v
        return 0

    lax.fori_loop(0, n // 16, body, 0)


@functools.partial(
    pl.kernel,
    out_type=jax.ShapeDtypeStruct((NC * N,), jnp.float32),
    mesh=_mesh,
    scratch_types=[
        pltpu.VMEM((DEG_CHUNKS, CHUNK), jnp.int32),
        pltpu.VMEM((CHUNK,), jnp.float32),
        pltpu.VMEM((N,), jnp.float32),
        pltpu.VMEM_SHARED((N,), jnp.float32),
    ],
)
def _deg_kernel(dst4_hbm, deg_hbm, dst_v, ones_v, zeros_v, acc_sh):
    c = lax.axis_index("c")
    s = lax.axis_index("s")
    w = c * NS + s

    pltpu.sync_copy(dst4_hbm.at[w], dst_v)
    _fill_1d(ones_v, CHUNK, 1.0)

    @pl.when(s == 0)
    def _():
        _fill_1d(zeros_v, N, 0.0)
        pltpu.sync_copy(zeros_v, acc_sh)

    plsc.subcore_barrier()

    def body(j, _):
        pltpu.sync_copy(ones_v, acc_sh.at[dst_v.at[j]], add=True)
        return 0

    lax.fori_loop(0, DEG_CHUNKS, body, 0)
    plsc.subcore_barrier()

    @pl.when(s == 0)
    def _():
        pltpu.sync_copy(acc_sh, zeros_v)
        pltpu.sync_copy(zeros_v, deg_hbm.at[pl.ds(c * N, N)])


@functools.partial(
    pl.kernel,
    out_type=jax.ShapeDtypeStruct((NC * N, DH), jnp.float32),
    mesh=_mesh,
    scratch_types=[
        pltpu.VMEM((CPP, CHUNK), jnp.int32),
        pltpu.VMEM((CPP, CHUNK), jnp.int32),
        pltpu.VMEM((CPP, CHUNK), jnp.int32),
        pltpu.VMEM((CPP, CHUNK), jnp.int32),
        pltpu.VMEM((CHUNK, DH), jnp.float32),
        pltpu.VMEM((CHUNK, DH), jnp.float32),
        pltpu.VMEM((CHUNK, DH), jnp.float32),
        pltpu.VMEM_SHARED((N, DH), jnp.float32),
        pltpu.SemaphoreType.DMA,
        pltpu.SemaphoreType.DMA,
        pltpu.SemaphoreType.DMA,
        pltpu.SemaphoreType.DMA,
    ],
)
def _segsum_kernel(h_hbm, src3_hbm, dst3_hbm, out_hbm,
                   srcA_v, dstA_v, srcB_v, dstB_v,
                   rows0_v, rows1_v, rows2_v, acc_sh,
                   gsem0, gsem1, gsem2, isem):
    c = lax.axis_index("c")
    s = lax.axis_index("s")
    w = c * NS + s

    NBUF = 3
    rows = (rows0_v, rows1_v, rows2_v)
    sems = (gsem0, gsem1, gsem2)

    pltpu.sync_copy(src3_hbm.at[w * SEG_PHASES], srcA_v)
    pltpu.sync_copy(dst3_hbm.at[s * SEG_PHASES], dstA_v)
    for b in range(NBUF - 1):
        pltpu.async_copy(h_hbm.at[srcA_v.at[b]], rows[b], sems[b])

    _fill_zeros_2d(rows2_v, CHUNK, DH)
    NZT = (N // CHUNK + NS - 1) // NS
    for t in range(NZT):
        j = s + t * NS

        @pl.when(j < N // CHUNK)
        def _(j=j):
            pltpu.async_copy(rows2_v, acc_sh.at[pl.ds(j * CHUNK, CHUNK)],
                             gsem2)
    for t in range(NZT):
        j = s + t * NS

        @pl.when(j < N // CHUNK)
        def _(j=j):
            pltpu.make_async_copy(
                rows2_v, acc_sh.at[pl.ds(j * CHUNK, CHUNK)], gsem2).wait()
    plsc.subcore_barrier()
    pltpu.async_copy(h_hbm.at[srcA_v.at[NBUF - 1]], rows[NBUF - 1],
                     sems[NBUF - 1])

    for p in range(SEG_PHASES):
        cs, cd = (srcA_v, dstA_v) if p % 2 == 0 else (srcB_v, dstB_v)
        ns, nd = (srcB_v, dstB_v) if p % 2 == 0 else (srcA_v, dstA_v)
        if p > 0:
            for b in range(NBUF):
                pltpu.async_copy(h_hbm.at[cs.at[b]], rows[b], sems[b])
        if p + 1 < SEG_PHASES:
            pltpu.async_copy(src3_hbm.at[w * SEG_PHASES + p + 1], ns, isem)
            pltpu.async_copy(dst3_hbm.at[s * SEG_PHASES + p + 1], nd, isem)

        def step(t, _, cs=cs, cd=cd):
            for b in range(NBUF):
                j = t * NBUF + b

                @pl.when(j < CPP)
                def _(j=j, b=b):
                    pltpu.make_async_copy(
                        h_hbm.at[cs.at[j]], rows[b], sems[b]).wait()
                    pltpu.sync_copy(rows[b], acc_sh.at[cd.at[j]], add=True)

                    @pl.when(j + NBUF < CPP)
                    def _():
                        pltpu.async_copy(
                            h_hbm.at[cs.at[j + NBUF]], rows[b], sems[b])
            return 0

        lax.fori_loop(0, (CPP + NBUF - 1) // NBUF, step, 0)
        if p + 1 < SEG_PHASES:
            pltpu.make_async_copy(
                src3_hbm.at[w * SEG_PHASES + p + 1], ns, isem).wait()
            pltpu.make_async_copy(
                dst3_hbm.at[s * SEG_PHASES + p + 1], nd, isem).wait()

    plsc.subcore_barrier()

    for t in range(NZT):
        j = s + t * NS
        b = t % 2

        @pl.when(j < N // CHUNK)
        def _(j=j, b=b, t=t):
            if t >= 2:
                jo = s + (t - 2) * NS
                pltpu.make_async_copy(
                    rows[b], out_hbm.at[pl.ds(c * N + jo * CHUNK, CHUNK)],
                    sems[b]).wait()
            pltpu.sync_copy(acc_sh.at[pl.ds(j * CHUNK, CHUNK)], rows[b])
            pltpu.async_copy(
                rows[b], out_hbm.at[pl.ds(c * N + j * CHUNK, CHUNK)], sems[b])
    for b in range(2):
        pltpu.make_async_copy(
            rows[b], out_hbm.at[pl.ds(c * N, CHUNK)], sems[b]).wait()


BN = 2000
NB = N // BN
_DOT = dict(preferred_element_type=jnp.float32)


def _l0_body(x_ref, w_ref, b_ref, dega_ref, degb_ref, h_ref, invd_ref):
    deg = dega_ref[...] + degb_ref[...]
    invd = lax.rsqrt(jnp.maximum(deg, 1.0))
    m = jnp.dot(x_ref[...], w_ref[...], **_DOT)
    h_ref[...] = (m + b_ref[...]) * invd
    invd_ref[...] = invd


def _l0_call(x, w0, b0, deg_a, deg_b):
    return pl.pallas_call(
        _l0_body,
        grid=(NB, NC),
        in_specs=[
            pl.BlockSpec((BN, D_IN), lambda i, c: (i, 0)),
            pl.BlockSpec((D_IN, DH), lambda i, c: (0, c)),
            pl.BlockSpec((1, DH), lambda i, c: (0, c)),
            pl.BlockSpec((BN, 1), lambda i, c: (i, 0)),
            pl.BlockSpec((BN, 1), lambda i, c: (i, 0)),
        ],
        out_specs=[
            pl.BlockSpec((BN, DH), lambda i, c: (c * NB + i, 0)),
            pl.BlockSpec((BN, 1), lambda i, c: (i, 0)),
        ],
        out_shape=[
            jax.ShapeDtypeStruct((NC * N, DH), jnp.float32),
            jax.ShapeDtypeStruct((N, 1), jnp.float32),
        ],
    )(x, w0, b0, deg_a, deg_b)


def _mid_body(sa_ref, sb_ref, invd_ref, w_ref, b_ref, h_ref):
    invd = invd_ref[...]
    ga = jnp.maximum(sa_ref[...] * invd, 0.0)
    gb = jnp.maximum(sb_ref[...] * invd, 0.0)
    m = (jnp.dot(ga, w_ref[0:DH, :], **_DOT)
         + jnp.dot(gb, w_ref[DH:D_HID, :], **_DOT))
    h_ref[...] = (m + b_ref[...]) * invd


def _mid_call(sarr, invd, w, b):
    return pl.pallas_call(
        _mid_body,
        grid=(NB, NC),
        in_specs=[
            pl.BlockSpec((BN, DH), lambda i, c: (i, 0)),
            pl.BlockSpec((BN, DH), lambda i, c: (NB + i, 0)),
            pl.BlockSpec((BN, 1), lambda i, c: (i, 0)),
            pl.BlockSpec((D_HID, DH), lambda i, c: (0, c)),
            pl.BlockSpec((1, DH), lambda i, c: (0, c)),
        ],
        out_specs=pl.BlockSpec((BN, DH), lambda i, c: (c * NB + i, 0)),
        out_shape=jax.ShapeDtypeStruct((NC * N, DH), jnp.float32),
    )(sarr, sarr, invd, w, b)


def _head_body(sa_ref, sb_ref, invd_ref, prompt_ref, wp1_ref, bp1_ref,
               wp2_ref, bp2_ref, ec_ref, out_ref):
    i = pl.program_id(0)
    invd = invd_ref[...]
    pa = sa_ref[...] * invd + prompt_ref[0:1, 0:DH]
    pb = sb_ref[...] * invd + prompt_ref[0:1, DH:D_HID]
    hh = jnp.maximum(jnp.dot(pa, wp1_ref[0:DH, :], **_DOT)
                     + jnp.dot(pb, wp1_ref[DH:D_HID, :], **_DOT)
                     + bp1_ref[...], 0.0)
    pi = jnp.dot(hh, wp2_ref[...], **_DOT) + bp2_ref[...]
    part = jnp.sum(jnp.abs(pi - ec_ref[...]), keepdims=True)
    acc = jnp.where(i == 0, jnp.zeros((1, 1), jnp.float32), out_ref[...]) + part
    out_ref[...] = jnp.where(i == NB - 1, acc / N, acc)


def _head_call(sarr, invd, prompt, wp1, bp1, wp2, bp2, ec2):
    return pl.pallas_call(
        _head_body,
        grid=(NB,),
        in_specs=[
            pl.BlockSpec((BN, DH), lambda i: (i, 0)),
            pl.BlockSpec((BN, DH), lambda i: (NB + i, 0)),
            pl.BlockSpec((BN, 1), lambda i: (i, 0)),
            pl.BlockSpec((1, D_HID), lambda i: (0, 0)),
            pl.BlockSpec((D_HID, D_HID), lambda i: (0, 0)),
            pl.BlockSpec((1, D_HID), lambda i: (0, 0)),
            pl.BlockSpec((D_HID, 1), lambda i: (0, 0)),
            pl.BlockSpec((1, 1), lambda i: (0, 0)),
            pl.BlockSpec((BN, 1), lambda i: (i, 0)),
        ],
        out_specs=pl.BlockSpec((1, 1), lambda i: (0, 0)),
        out_shape=jax.ShapeDtypeStruct((1, 1), jnp.float32),
    )(sarr, sarr, invd, prompt, wp1, bp1, wp2, bp2, ec2)


def kernel(x, edge_index, edge_attr, eigenvector_centrality, batch,
           W0, b0, W1, b1, W2, b2, W3, b3, W4, b4,
           Wp1, bp1, Wp2, bp2, prompt):
    src = edge_index[0]
    dst = edge_index[1]
    src2 = jnp.concatenate([src, src + N])
    src3 = src2.reshape(NC * NS * SEG_PHASES, CPP, CHUNK)
    dst3 = dst.reshape(NS * SEG_PHASES, CPP, CHUNK)
    dst4 = dst.reshape(NC * NS, DEG_CHUNKS, CHUNK)

    deg2 = _deg_kernel(dst4)
    deg_a = deg2[:N].reshape(N, 1)
    deg_b = deg2[N:].reshape(N, 1)

    Ws = [W1, W2, W3, W4]
    bs = [b1.reshape(1, D_HID), b2.reshape(1, D_HID),
          b3.reshape(1, D_HID), b4.reshape(1, D_HID)]

    h, invd = _l0_call(x, W0, b0.reshape(1, D_HID), deg_a, deg_b)
    for i in range(NUM_LAYER - 1):
        sarr = _segsum_kernel(h, src3, dst3)
        h = _mid_call(sarr, invd, Ws[i], bs[i])
    sarr = _segsum_kernel(h, src3, dst3)

    out = _head_call(sarr, invd, prompt, Wp1, bp1.reshape(1, D_HID),
                     Wp2, bp2.reshape(1, 1),
                     eigenvector_centrality.reshape(N, 1))
    return out.reshape(())


# speedup vs baseline: 1.0189x; 1.0189x over previous
import functools

import jax
import jax.numpy as jnp
from jax import lax
from jax.experimental import pallas as pl
from jax.experimental.pallas import tpu as pltpu
from jax.experimental.pallas import tpu_sc as plsc

N = 10000
E = 320000
D_IN = 128
D_HID = 256
DH = 128
NUM_LAYER = 5
NC = 2
NS = 16
CHUNK = 80
SEG_CHUNKS = E // NS // CHUNK
SEG_PHASES = 5
CPP = SEG_CHUNKS // SEG_PHASES
DEG_CHUNKS = E // (NC * NS) // CHUNK
ZR = 125

_mesh = plsc.VectorSubcoreMesh(
    core_axis_name="c", subcore_axis_name="s", num_cores=NC, num_subcores=NS)


def _fill_zeros_2d(ref, rows, cols):
    z = jnp.zeros((16,), jnp.float32)

    def row_body(r, _):
        def col_body(j, _):
            ref[r, pl.ds(j * 16, 16)] = z
            return 0
        return lax.fori_loop(0, cols // 16, col_body, 0)

    lax.fori_loop(0, rows, row_body, 0)


def _fill_1d(ref, n, value):
    v = jnp.full((16,), value, jnp.float32)

    def body(j, _):
        ref[pl.ds(j * 16, 16)] = v
        return 0

    lax.fori_loop(0, n // 16, body, 0)


@functools.partial(
    pl.kernel,
    out_type=jax.ShapeDtypeStruct((NC * N,), jnp.float32),
    mesh=_mesh,
    scratch_types=[
        pltpu.VMEM((DEG_CHUNKS, CHUNK), jnp.int32),
        pltpu.VMEM((CHUNK,), jnp.float32),
        pltpu.VMEM((N,), jnp.float32),
        pltpu.VMEM_SHARED((N,), jnp.float32),
    ],
)
def _deg_kernel(dst4_hbm, deg_hbm, dst_v, ones_v, zeros_v, acc_sh):
    c = lax.axis_index("c")
    s = lax.axis_index("s")
    w = c * NS + s

    pltpu.sync_copy(dst4_hbm.at[w], dst_v)
    _fill_1d(ones_v, CHUNK, 1.0)

    @pl.when(s == 0)
    def _():
        _fill_1d(zeros_v, N, 0.0)
        pltpu.sync_copy(zeros_v, acc_sh)

    plsc.subcore_barrier()

    def body(j, _):
        pltpu.sync_copy(ones_v, acc_sh.at[dst_v.at[j]], add=True)
        return 0

    lax.fori_loop(0, DEG_CHUNKS, body, 0)
    plsc.subcore_barrier()

    @pl.when(s == 0)
    def _():
        pltpu.sync_copy(acc_sh, zeros_v)
        pltpu.sync_copy(zeros_v, deg_hbm.at[pl.ds(c * N, N)])


@functools.partial(
    pl.kernel,
    out_type=jax.ShapeDtypeStruct((NC * N, DH), jnp.float32),
    mesh=_mesh,
    scratch_types=[
        pltpu.VMEM((CPP, CHUNK), jnp.int32),
        pltpu.VMEM((CPP, CHUNK), jnp.int32),
        pltpu.VMEM((CHUNK, DH), jnp.float32),
        pltpu.VMEM((CHUNK, DH), jnp.float32),
        pltpu.VMEM((CHUNK, DH), jnp.float32),
        pltpu.VMEM_SHARED((N, DH), jnp.float32),
        pltpu.SemaphoreType.DMA,
        pltpu.SemaphoreType.DMA,
        pltpu.SemaphoreType.DMA,
    ],
)
def _segsum_kernel(h_hbm, src3_hbm, dst3_hbm, out_hbm,
                   src_v, dst_v, rows0_v, rows1_v, rows2_v, acc_sh,
                   gsem0, gsem1, gsem2):
    c = lax.axis_index("c")
    s = lax.axis_index("s")
    w = c * NS + s

    NBUF = 3
    rows = (rows0_v, rows1_v, rows2_v)
    sems = (gsem0, gsem1, gsem2)

    pltpu.sync_copy(src3_hbm.at[w * SEG_PHASES], src_v)
    pltpu.sync_copy(dst3_hbm.at[s * SEG_PHASES], dst_v)
    for b in range(NBUF - 1):
        pltpu.async_copy(h_hbm.at[src_v.at[b]], rows[b], sems[b])

    _fill_zeros_2d(rows2_v, CHUNK, DH)
    NZT = (N // CHUNK + NS - 1) // NS
    for t in range(NZT):
        j = s + t * NS

        @pl.when(j < N // CHUNK)
        def _(j=j):
            pltpu.async_copy(rows2_v, acc_sh.at[pl.ds(j * CHUNK, CHUNK)],
                             gsem2)
    for t in range(NZT):
        j = s + t * NS

        @pl.when(j < N // CHUNK)
        def _(j=j):
            pltpu.make_async_copy(
                rows2_v, acc_sh.at[pl.ds(j * CHUNK, CHUNK)], gsem2).wait()
    plsc.subcore_barrier()
    pltpu.async_copy(h_hbm.at[src_v.at[NBUF - 1]], rows[NBUF - 1],
                     sems[NBUF - 1])

    for p in range(SEG_PHASES):
        if p > 0:
            pltpu.sync_copy(src3_hbm.at[w * SEG_PHASES + p], src_v)
            pltpu.sync_copy(dst3_hbm.at[s * SEG_PHASES + p], dst_v)
            for b in range(NBUF):
                pltpu.async_copy(h_hbm.at[src_v.at[b]], rows[b], sems[b])

        def step(t, _):
            for b in range(NBUF):
                j = t * NBUF + b

                @pl.when(j < CPP)
                def _(j=j, b=b):
                    pltpu.make_async_copy(
                        h_hbm.at[src_v.at[j]], rows[b], sems[b]).wait()
                    pltpu.sync_copy(rows[b], acc_sh.at[dst_v.at[j]], add=True)

                    @pl.when(j + NBUF < CPP)
                    def _():
                        pltpu.async_copy(
                            h_hbm.at[src_v.at[j + NBUF]], rows[b], sems[b])
            return 0

        lax.fori_loop(0, (CPP + NBUF - 1) // NBUF, step, 0)

    plsc.subcore_barrier()

    for t in range(NZT):
        j = s + t * NS
        b = t % 2

        @pl.when(j < N // CHUNK)
        def _(j=j, b=b, t=t):
            if t >= 2:
                jo = s + (t - 2) * NS
                pltpu.make_async_copy(
                    rows[b], out_hbm.at[pl.ds(c * N + jo * CHUNK, CHUNK)],
                    sems[b]).wait()
            pltpu.sync_copy(acc_sh.at[pl.ds(j * CHUNK, CHUNK)], rows[b])
            pltpu.async_copy(
                rows[b], out_hbm.at[pl.ds(c * N + j * CHUNK, CHUNK)], sems[b])
    for b in range(2):
        pltpu.make_async_copy(
            rows[b], out_hbm.at[pl.ds(c * N, CHUNK)], sems[b]).wait()


BN = 2000
NB = N // BN
_DOT = dict(preferred_element_type=jnp.float32)


def _l0_body(x_ref, w_ref, b_ref, dega_ref, degb_ref, h_ref, invd_ref):
    deg = dega_ref[...] + degb_ref[...]
    invd = lax.rsqrt(jnp.maximum(deg, 1.0))
    m = jnp.dot(x_ref[...], w_ref[...], **_DOT)
    h_ref[...] = (m + b_ref[...]) * invd
    invd_ref[...] = invd


def _l0_call(x, w0, b0, deg_a, deg_b):
    return pl.pallas_call(
        _l0_body,
        grid=(NB, NC),
        in_specs=[
            pl.BlockSpec((BN, D_IN), lambda i, c: (i, 0)),
            pl.BlockSpec((D_IN, DH), lambda i, c: (0, c)),
            pl.BlockSpec((1, DH), lambda i, c: (0, c)),
            pl.BlockSpec((BN, 1), lambda i, c: (i, 0)),
            pl.BlockSpec((BN, 1), lambda i, c: (i, 0)),
        ],
        out_specs=[
            pl.BlockSpec((BN, DH), lambda i, c: (c * NB + i, 0)),
            pl.BlockSpec((BN, 1), lambda i, c: (i, 0)),
        ],
        out_shape=[
            jax.ShapeDtypeStruct((NC * N, DH), jnp.float32),
            jax.ShapeDtypeStruct((N, 1), jnp.float32),
        ],
    )(x, w0, b0, deg_a, deg_b)


def _mid_body(sa_ref, sb_ref, invd_ref, w_ref, b_ref, h_ref):
    invd = invd_ref[...]
    ga = jnp.maximum(sa_ref[...] * invd, 0.0)
    gb = jnp.maximum(sb_ref[...] * invd, 0.0)
    m = (jnp.dot(ga, w_ref[0:DH, :], **_DOT)
         + jnp.dot(gb, w_ref[DH:D_HID, :], **_DOT))
    h_ref[...] = (m + b_ref[...]) * invd


def _mid_call(sarr, invd, w, b):
    return pl.pallas_call(
        _mid_body,
        grid=(NB, NC),
        in_specs=[
            pl.BlockSpec((BN, DH), lambda i, c: (i, 0)),
            pl.BlockSpec((BN, DH), lambda i, c: (NB + i, 0)),
            pl.BlockSpec((BN, 1), lambda i, c: (i, 0)),
            pl.BlockSpec((D_HID, DH), lambda i, c: (0, c)),
            pl.BlockSpec((1, DH), lambda i, c: (0, c)),
        ],
        out_specs=pl.BlockSpec((BN, DH), lambda i, c: (c * NB + i, 0)),
        out_shape=jax.ShapeDtypeStruct((NC * N, DH), jnp.float32),
    )(sarr, sarr, invd, w, b)


def _head_body(sa_ref, sb_ref, invd_ref, prompt_ref, wp1_ref, bp1_ref,
               wp2_ref, bp2_ref, ec_ref, out_ref):
    i = pl.program_id(0)
    invd = invd_ref[...]
    pa = sa_ref[...] * invd + prompt_ref[0:1, 0:DH]
    pb = sb_ref[...] * invd + prompt_ref[0:1, DH:D_HID]
    hh = jnp.maximum(jnp.dot(pa, wp1_ref[0:DH, :], **_DOT)
                     + jnp.dot(pb, wp1_ref[DH:D_HID, :], **_DOT)
                     + bp1_ref[...], 0.0)
    pi = jnp.dot(hh, wp2_ref[...], **_DOT) + bp2_ref[...]
    part = jnp.sum(jnp.abs(pi - ec_ref[...]), keepdims=True)
    acc = jnp.where(i == 0, jnp.zeros((1, 1), jnp.float32), out_ref[...]) + part
    out_ref[...] = jnp.where(i == NB - 1, acc / N, acc)


def _head_call(sarr, invd, prompt, wp1, bp1, wp2, bp2, ec2):
    return pl.pallas_call(
        _head_body,
        grid=(NB,),
        in_specs=[
            pl.BlockSpec((BN, DH), lambda i: (i, 0)),
            pl.BlockSpec((BN, DH), lambda i: (NB + i, 0)),
            pl.BlockSpec((BN, 1), lambda i: (i, 0)),
            pl.BlockSpec((1, D_HID), lambda i: (0, 0)),
            pl.BlockSpec((D_HID, D_HID), lambda i: (0, 0)),
            pl.BlockSpec((1, D_HID), lambda i: (0, 0)),
            pl.BlockSpec((D_HID, 1), lambda i: (0, 0)),
            pl.BlockSpec((1, 1), lambda i: (0, 0)),
            pl.BlockSpec((BN, 1), lambda i: (i, 0)),
        ],
        out_specs=pl.BlockSpec((1, 1), lambda i: (0, 0)),
        out_shape=jax.ShapeDtypeStruct((1, 1), jnp.float32),
    )(sarr, sarr, invd, prompt, wp1, bp1, wp2, bp2, ec2)


def kernel(x, edge_index, edge_attr, eigenvector_centrality, batch,
           W0, b0, W1, b1, W2, b2, W3, b3, W4, b4,
           Wp1, bp1, Wp2, bp2, prompt):
    src = edge_index[0]
    dst = edge_index[1]
    src2 = jnp.concatenate([src, src + N])
    src3 = src2.reshape(NC * NS * SEG_PHASES, CPP, CHUNK)
    dst3 = dst.reshape(NS * SEG_PHASES, CPP, CHUNK)
    dst4 = dst.reshape(NC * NS, DEG_CHUNKS, CHUNK)

    deg2 = _deg_kernel(dst4)
    deg_a = deg2[:N].reshape(N, 1)
    deg_b = deg2[N:].reshape(N, 1)

    Ws = [W1, W2, W3, W4]
    bs = [b1.reshape(1, D_HID), b2.reshape(1, D_HID),
          b3.reshape(1, D_HID), b4.reshape(1, D_HID)]

    h, invd = _l0_call(x, W0, b0.reshape(1, D_HID), deg_a, deg_b)
    for i in range(NUM_LAYER - 1):
        sarr = _segsum_kernel(h, src3, dst3)
        h = _mid_call(sarr, invd, Ws[i], bs[i])
    sarr = _segsum_kernel(h, src3, dst3)

    out = _head_call(sarr, invd, prompt, Wp1, bp1.reshape(1, D_HID),
                     Wp2, bp2.reshape(1, 1),
                     eigenvector_centrality.reshape(N, 1))
    return out.reshape(())
